# Initial kernel scaffold; baseline (speedup 1.0000x reference)
import jax
import jax.numpy as jnp
from jax.experimental import pallas as pl


def kernel(x, edge_index, W1, b1, g1, be1, W2, b2, g2, be2, W3, b3, g3, be3, W4, b4, a):
    raise NotImplementedError("write your pallas kernel here")


# speedup vs baseline: 14.7087x; 14.7087x over previous
import functools

import jax
import jax.numpy as jnp
from jax import lax
from jax.experimental import pallas as pl
from jax.experimental.pallas import tpu as pltpu
from jax.experimental.pallas import tpu_sc as plsc

N = 10000
E = 320000
D_OUT = 40

NPAD = 10240
ROWS_PER_TILE = NPAD // 16
CH = 128
NW = 32
NCHUNKS = 79
EPT = NCHUNKS * CH
EPAD = NW * EPT


def _vector_mesh():
    return plsc.VectorSubcoreMesh(core_axis_name="c", subcore_axis_name="s")


_SC_PARAMS = pltpu.CompilerParams(use_tc_tiling_on_sc=False)


def _sc_degree(dst_p):
    @functools.partial(
        pl.kernel,
        out_type=jax.ShapeDtypeStruct((2, NPAD), jnp.float32),
        mesh=_vector_mesh(),
        scratch_types=[
            pltpu.VMEM((CH,), jnp.int32),
            pltpu.VMEM((CH,), jnp.float32),
            pltpu.VMEM((ROWS_PER_TILE,), jnp.float32),
            pltpu.VMEM_SHARED((NPAD,), jnp.float32),
        ],
    )
    def k(dst_hbm, out_hbm, idx_v, ones_v, zbuf_v, acc):
        cid = lax.axis_index("c")
        sid = lax.axis_index("s")

        @pl.loop(0, CH, step=16)
        def _(i):
            ones_v[pl.ds(i, 16)] = jnp.ones((16,), jnp.float32)

        @pl.loop(0, ROWS_PER_TILE, step=16)
        def _(i):
            zbuf_v[pl.ds(i, 16)] = jnp.zeros((16,), jnp.float32)

        pltpu.sync_copy(zbuf_v, acc.at[pl.ds(sid * ROWS_PER_TILE, ROWS_PER_TILE)])
        plsc.subcore_barrier()

        w = cid * 16 + sid

        @pl.loop(0, NCHUNKS)
        def _(kk):
            base = w * EPT + kk * CH
            pltpu.sync_copy(dst_hbm.at[pl.ds(base, CH)], idx_v)
            pltpu.sync_copy(ones_v, acc.at[idx_v], add=True)

        plsc.subcore_barrier()
        pltpu.sync_copy(
            acc.at[pl.ds(sid * ROWS_PER_TILE, ROWS_PER_TILE)],
            out_hbm.at[cid, pl.ds(sid * ROWS_PER_TILE, ROWS_PER_TILE)],
        )

    return k(dst_p)


def _sc_aggregate(y_pad, src_p, dst_p):
    d = y_pad.shape[1]

    @functools.partial(
        pl.kernel,
        out_type=jax.ShapeDtypeStruct((2, NPAD, d), jnp.float32),
        mesh=_vector_mesh(),
        scratch_types=[
            pltpu.VMEM((CH,), jnp.int32),
            pltpu.VMEM((CH,), jnp.int32),
            pltpu.VMEM((CH, d), jnp.float32),
            pltpu.VMEM_SHARED((NPAD, d), jnp.float32),
        ],
        compiler_params=_SC_PARAMS,
    )
    def k(y_hbm, src_hbm, dst_hbm, out_hbm, src_v, dst_v, rows_v, acc):
        cid = lax.axis_index("c")
        sid = lax.axis_index("s")

        @pl.loop(0, CH)
        def _(i):
            @pl.loop(0, d, step=16)
            def _(j):
                rows_v[i, pl.ds(j, 16)] = jnp.zeros((16,), jnp.float32)

        row0 = sid * ROWS_PER_TILE
        for b in range(ROWS_PER_TILE // CH):
            pltpu.sync_copy(rows_v, acc.at[pl.ds(row0 + b * CH, CH)])
        plsc.subcore_barrier()

        w = cid * 16 + sid

        @pl.loop(0, NCHUNKS)
        def _(kk):
            base = w * EPT + kk * CH
            pltpu.sync_copy(src_hbm.at[pl.ds(base, CH)], src_v)
            pltpu.sync_copy(dst_hbm.at[pl.ds(base, CH)], dst_v)
            pltpu.sync_copy(y_hbm.at[src_v], rows_v)
            pltpu.sync_copy(rows_v, acc.at[dst_v], add=True)

        plsc.subcore_barrier()
        for b in range(ROWS_PER_TILE // CH):
            r0 = row0 + b * CH
            pltpu.sync_copy(acc.at[pl.ds(r0, CH)], out_hbm.at[cid, pl.ds(r0, CH)])

    return k(y_pad, src_p, dst_p)


def _tc_matmul(x, w):
    def body(x_ref, w_ref, o_ref):
        o_ref[...] = jnp.dot(x_ref[...], w_ref[...], preferred_element_type=jnp.float32)

    return pl.pallas_call(
        body, out_shape=jax.ShapeDtypeStruct((x.shape[0], w.shape[1]), jnp.float32)
    )(x, w)


def _tc_prep(h1, deg2):
    def body(h_ref, deg_ref, dinv_ref, y_ref):
        deg = deg_ref[0] + deg_ref[1] + 1.0
        dinv = lax.rsqrt(deg)
        dinv_ref[...] = dinv
        y_ref[pl.ds(0, N), :] = dinv[:N] * h_ref[...]
        y_ref[pl.ds(N, NPAD - N), :] = jnp.zeros((NPAD - N, h_ref.shape[1]), jnp.float32)

    return pl.pallas_call(
        body,
        out_shape=[
            jax.ShapeDtypeStruct((NPAD, 1), jnp.float32),
            jax.ShapeDtypeStruct((NPAD, h1.shape[1]), jnp.float32),
        ],
    )(h1, deg2)


def _tc_stage(z, y, dinvp, b, g, be, a2, w):
    dn = w.shape[1]

    def body(z_ref, y_ref, dinv_ref, b_ref, g_ref, be_ref, a_ref, w_ref, o_ref):
        zz = z_ref[0, :N, :] + z_ref[1, :N, :]
        dinv = dinv_ref[:N, :]
        t = dinv * (zz + y_ref[:N, :]) + b_ref[...]
        m = jnp.mean(t, axis=0, keepdims=True)
        v = jnp.mean((t - m) ** 2, axis=0, keepdims=True)
        t = (t - m) * lax.rsqrt(v + 1e-5) * g_ref[...] + be_ref[...]
        t = jnp.where(t >= 0, t, a_ref[0, 0] * t)
        h = jnp.dot(t, w_ref[...], preferred_element_type=jnp.float32)
        o_ref[pl.ds(0, N), :] = dinv * h
        o_ref[pl.ds(N, NPAD - N), :] = jnp.zeros((NPAD - N, dn), jnp.float32)

    return pl.pallas_call(
        body, out_shape=jax.ShapeDtypeStruct((NPAD, dn), jnp.float32)
    )(z, y, dinvp, b, g, be, a2, w)


def _tc_final(z, y, dinvp, b):
    def body(z_ref, y_ref, dinv_ref, b_ref, o_ref):
        zz = z_ref[0, :N, :] + z_ref[1, :N, :]
        t = dinv_ref[:N, :] * (zz + y_ref[:N, :]) + b_ref[...]
        t = t[:, :D_OUT]
        m = jnp.max(t, axis=1, keepdims=True)
        s = jnp.log(jnp.sum(jnp.exp(t - m), axis=1, keepdims=True))
        o_ref[...] = t - m - s

    return pl.pallas_call(
        body, out_shape=jax.ShapeDtypeStruct((N, D_OUT), jnp.float32)
    )(z, y, dinvp, b)


def kernel(x, edge_index, W1, b1, g1, be1, W2, b2, g2, be2, W3, b3, g3, be3, W4, b4, a):
    src = edge_index[0].astype(jnp.int32)
    dst = edge_index[1].astype(jnp.int32)
    pad_idx = N + (jnp.arange(EPAD - E, dtype=jnp.int32) % (NPAD - N))
    src_p = jnp.concatenate([src, pad_idx])
    dst_p = jnp.concatenate([dst, pad_idx])

    a2 = a.reshape(1, 1)
    b1r, g1r, be1r = b1.reshape(1, -1), g1.reshape(1, -1), be1.reshape(1, -1)
    b2r, g2r, be2r = b2.reshape(1, -1), g2.reshape(1, -1), be2.reshape(1, -1)
    b3r, g3r, be3r = b3.reshape(1, -1), g3.reshape(1, -1), be3.reshape(1, -1)
    W4p = jnp.pad(W4, ((0, 0), (0, 8)))
    b4p = jnp.pad(b4, (0, 8)).reshape(1, -1)

    deg2 = _sc_degree(dst_p).reshape(2, NPAD, 1)
    h1 = _tc_matmul(x, W1)
    dinvp, y1 = _tc_prep(h1, deg2)

    z1 = _sc_aggregate(y1, src_p, dst_p)
    y2 = _tc_stage(z1, y1, dinvp, b1r, g1r, be1r, a2, W2)
    z2 = _sc_aggregate(y2, src_p, dst_p)
    y3 = _tc_stage(z2, y2, dinvp, b2r, g2r, be2r, a2, W3)
    z3 = _sc_aggregate(y3, src_p, dst_p)
    y4 = _tc_stage(z3, y3, dinvp, b3r, g3r, be3r, a2, W4p)
    z4 = _sc_aggregate(y4, src_p, dst_p)
    return _tc_final(z4, y4, dinvp, b4p)


# speedup vs baseline: 33.5037x; 2.2778x over previous
import functools

import jax
import jax.numpy as jnp
from jax import lax
from jax.experimental import pallas as pl
from jax.experimental.pallas import tpu as pltpu
from jax.experimental.pallas import tpu_sc as plsc

N = 10000
E = 320000
D_OUT = 40

NPAD = 10240
ROWS_PER_TILE = NPAD // 16
CH = 128
NW = 32
SCH = 512
NSUP = 20
EPT = NSUP * SCH
EPAD = NW * EPT
L = N // 2
LPAD = NPAD // 2


def _vector_mesh():
    return plsc.VectorSubcoreMesh(core_axis_name="c", subcore_axis_name="s")


_SC_PARAMS = pltpu.CompilerParams(use_tc_tiling_on_sc=False)


def _sc_degree(dst_p):
    @functools.partial(
        pl.kernel,
        out_type=jax.ShapeDtypeStruct((2, NPAD), jnp.float32),
        mesh=_vector_mesh(),
        scratch_types=[
            pltpu.VMEM((NSUP, SCH), jnp.int32),
            pltpu.VMEM((SCH,), jnp.float32),
            pltpu.VMEM((ROWS_PER_TILE,), jnp.float32),
            pltpu.VMEM_SHARED((NPAD,), jnp.float32),
        ],
        compiler_params=_SC_PARAMS,
    )
    def k(dst_hbm, out_hbm, idx_v, ones_v, zbuf_v, acc):
        cid = lax.axis_index("c")
        sid = lax.axis_index("s")

        @pl.loop(0, SCH, step=16)
        def _(i):
            ones_v[pl.ds(i, 16)] = jnp.ones((16,), jnp.float32)

        @pl.loop(0, ROWS_PER_TILE, step=16)
        def _(i):
            zbuf_v[pl.ds(i, 16)] = jnp.zeros((16,), jnp.float32)

        pltpu.sync_copy(zbuf_v, acc.at[pl.ds(sid * ROWS_PER_TILE, ROWS_PER_TILE)])

        w = cid * 16 + sid
        pltpu.sync_copy(dst_hbm.at[pl.ds(w * NSUP, NSUP)], idx_v)
        plsc.subcore_barrier()

        @pl.loop(0, NSUP)
        def _(kk):
            pltpu.sync_copy(ones_v, acc.at[idx_v.at[kk]], add=True)

        plsc.subcore_barrier()
        pltpu.sync_copy(
            acc.at[pl.ds(sid * ROWS_PER_TILE, ROWS_PER_TILE)],
            out_hbm.at[cid, pl.ds(sid * ROWS_PER_TILE, ROWS_PER_TILE)],
        )

    return k(dst_p)


def _sc_aggregate(y_view, src_p, dst_p, d, idx_mul=1, idx_off=0):
    @functools.partial(
        pl.kernel,
        out_type=jax.ShapeDtypeStruct((2, NPAD, d), jnp.float32),
        mesh=_vector_mesh(),
        scratch_types=[
            pltpu.VMEM((NSUP, SCH), jnp.int32),
            pltpu.VMEM((NSUP, SCH), jnp.int32),
            pltpu.VMEM((SCH, d), jnp.float32),
            pltpu.VMEM((SCH, d), jnp.float32),
            pltpu.SemaphoreType.DMA,
            pltpu.SemaphoreType.DMA,
            pltpu.VMEM_SHARED((NPAD, d), jnp.float32),
        ],
        compiler_params=_SC_PARAMS,
    )
    def k(y_hbm, src_hbm, dst_hbm, out_hbm, srcs_v, dsts_v, bigA, bigB, sA, sB, acc):
        bufs = (bigA, bigB)
        sems = (sA, sB)
        cid = lax.axis_index("c")
        sid = lax.axis_index("s")

        @pl.loop(0, CH)
        def _(i):
            @pl.loop(0, d, step=16)
            def _(j):
                bigA[i, pl.ds(j, 16)] = jnp.zeros((16,), jnp.float32)

        row0 = sid * ROWS_PER_TILE
        for b in range(ROWS_PER_TILE // CH):
            pltpu.sync_copy(bigA.at[pl.ds(0, CH)], acc.at[pl.ds(row0 + b * CH, CH)])

        w = cid * 16 + sid
        pltpu.sync_copy(src_hbm.at[pl.ds(w * NSUP, NSUP)], srcs_v)
        pltpu.sync_copy(dst_hbm.at[pl.ds(w * NSUP, NSUP)], dsts_v)
        if idx_mul != 1 or idx_off != 0:
            @pl.loop(0, NSUP)
            def _(i):
                @pl.loop(0, SCH, step=16)
                def _(j):
                    srcs_v[i, pl.ds(j, 16)] = (
                        srcs_v[i, pl.ds(j, 16)] * idx_mul + idx_off
                    )
        plsc.subcore_barrier()

        for j in range(2):
            pltpu.sync_copy(y_hbm.at[srcs_v.at[j]], bufs[j])
            pltpu.async_copy(bufs[j], acc.at[dsts_v.at[j]], sems[j], add=True)

        @pl.loop(2, NSUP, step=2)
        def _(c):
            for j in range(2):
                pltpu.make_async_copy(bufs[j], acc.at[dsts_v.at[0]], sems[j]).wait()
                pltpu.sync_copy(y_hbm.at[srcs_v.at[c + j]], bufs[j])
                pltpu.async_copy(bufs[j], acc.at[dsts_v.at[c + j]], sems[j], add=True)

        for j in range(2):
            pltpu.make_async_copy(bufs[j], acc.at[dsts_v.at[0]], sems[j]).wait()

        plsc.subcore_barrier()
        pltpu.sync_copy(
            acc.at[pl.ds(row0, ROWS_PER_TILE)],
            out_hbm.at[cid, pl.ds(row0, ROWS_PER_TILE)],
        )

    return k(y_view, src_p, dst_p)


def _tc_mm_pair(xp, w):
    din = w.shape[0]
    dn = w.shape[1]

    def body(x_ref, w_ref, o_ref):
        xv = x_ref[...]
        ww = w_ref[...]
        hE = jnp.dot(xv[:, :din], ww, preferred_element_type=jnp.float32)
        hO = jnp.dot(xv[:, din:], ww, preferred_element_type=jnp.float32)
        o_ref[...] = jnp.concatenate([hE, hO], axis=1)

    return pl.pallas_call(
        body, out_shape=jax.ShapeDtypeStruct((xp.shape[0], 2 * dn), jnp.float32)
    )(xp, w)


def _tc_prep(hp, degl):
    w2 = hp.shape[1]

    def body(h_ref, deg_ref, dl_ref, y_ref):
        ds_ = deg_ref[0] + deg_ref[1] + 1.0
        dl = lax.rsqrt(ds_)
        dl_ref[...] = dl
        hv = h_ref[...]
        half = w2 // 2
        yE = dl[:L, 0:1] * hv[:, :half]
        yO = dl[:L, 1:2] * hv[:, half:]
        y_ref[pl.ds(0, L), :] = jnp.concatenate([yE, yO], axis=1)
        y_ref[pl.ds(L, LPAD - L), :] = jnp.zeros((LPAD - L, w2), jnp.float32)

    return pl.pallas_call(
        body,
        out_shape=[
            jax.ShapeDtypeStruct((LPAD, 2), jnp.float32),
            jax.ShapeDtypeStruct((LPAD, w2), jnp.float32),
        ],
    )(hp, degl)


def _tc_stage(zs, y, dl, b, g, be, a2, w):
    nz = len(zs)
    din = w.shape[0]
    dn = w.shape[1]

    def body(*refs):
        z_refs = refs[:nz]
        y_ref, dl_ref, b_ref, g_ref, be_ref, a_ref, w_ref, o_ref = refs[nz:]
        zsums = [zr[0] + zr[1] for zr in z_refs]
        if nz == 1:
            zE = zsums[0][:L, :64]
            zO = zsums[0][:L, 64:]
        else:
            zE = jnp.concatenate([zp[:L, :64] for zp in zsums], axis=1)
            zO = jnp.concatenate([zp[:L, 64:] for zp in zsums], axis=1)
        yv = y_ref[...]
        half = yv.shape[1] // 2
        dlv = dl_ref[...]
        dE = dlv[:L, 0:1]
        dO = dlv[:L, 1:2]
        bb = b_ref[...]
        tE = dE * (zE + yv[:L, :half]) + bb
        tO = dO * (zO + yv[:L, half:]) + bb
        m = (jnp.sum(tE, 0, keepdims=True) + jnp.sum(tO, 0, keepdims=True)) / N
        s2 = (jnp.sum(tE * tE, 0, keepdims=True)
              + jnp.sum(tO * tO, 0, keepdims=True)) / N
        rs = lax.rsqrt(s2 - m * m + 1e-5) * g_ref[...]
        bev = be_ref[...]
        av = a_ref[0, 0]
        tE = (tE - m) * rs + bev
        tO = (tO - m) * rs + bev
        tE = jnp.where(tE >= 0, tE, av * tE)
        tO = jnp.where(tO >= 0, tO, av * tO)
        ww = w_ref[...]
        hE = jnp.dot(tE, ww, preferred_element_type=jnp.float32)
        hO = jnp.dot(tO, ww, preferred_element_type=jnp.float32)
        yn = jnp.concatenate([dE * hE, dO * hO], axis=1)
        o_ref[pl.ds(0, L), :] = yn
        o_ref[pl.ds(L, LPAD - L), :] = jnp.zeros((LPAD - L, 2 * dn), jnp.float32)

    return pl.pallas_call(
        body, out_shape=jax.ShapeDtypeStruct((LPAD, 2 * dn), jnp.float32)
    )(*zs, y, dl, b, g, be, a2, w)


def _tc_final(z, y, dl, b):
    def body(z_ref, y_ref, dl_ref, b_ref, o_ref):
        zp = z_ref[0] + z_ref[1]
        yv = y_ref[...]
        dlv = dl_ref[...]
        bb = b_ref[...]
        tE = dlv[:L, 0:1] * (zp[:L, :64] + yv[:L, :64]) + bb
        tO = dlv[:L, 1:2] * (zp[:L, 64:] + yv[:L, 64:]) + bb
        tE = tE[:, :D_OUT]
        tO = tO[:, :D_OUT]
        mE = jnp.max(tE, axis=1, keepdims=True)
        mO = jnp.max(tO, axis=1, keepdims=True)
        sE = jnp.log(jnp.sum(jnp.exp(tE - mE), axis=1, keepdims=True))
        sO = jnp.log(jnp.sum(jnp.exp(tO - mO), axis=1, keepdims=True))
        o_ref[...] = jnp.concatenate([tE - mE - sE, tO - mO - sO], axis=1)

    return pl.pallas_call(
        body, out_shape=jax.ShapeDtypeStruct((L, 2 * D_OUT), jnp.float32)
    )(z, y, dl, b)


def kernel(x, edge_index, W1, b1, g1, be1, W2, b2, g2, be2, W3, b3, g3, be3, W4, b4, a):
    src = edge_index[0].astype(jnp.int32)
    dst = edge_index[1].astype(jnp.int32)
    pad_idx = N + (jnp.arange(EPAD - E, dtype=jnp.int32) % (NPAD - N))
    src_p = jnp.concatenate([src, pad_idx]).reshape(NW * NSUP, SCH)
    dst_p = jnp.concatenate([dst, pad_idx]).reshape(NW * NSUP, SCH)

    a2 = a.reshape(1, 1)
    b1r, g1r, be1r = b1.reshape(1, -1), g1.reshape(1, -1), be1.reshape(1, -1)
    b2r, g2r, be2r = b2.reshape(1, -1), g2.reshape(1, -1), be2.reshape(1, -1)
    W3p = jnp.pad(W3, ((0, 0), (0, 32)))
    b3p = jnp.pad(b3, (0, 32)).reshape(1, -1)
    g3p = jnp.pad(g3, (0, 32)).reshape(1, -1)
    be3p = jnp.pad(be3, (0, 32)).reshape(1, -1)
    W4p = jnp.pad(W4, ((0, 32), (0, 24)))
    b4p = jnp.pad(b4, (0, 24)).reshape(1, -1)

    xp = x.reshape(L, 256)
    degl = _sc_degree(dst_p).reshape(2, LPAD, 2)
    hp = _tc_mm_pair(xp, W1)
    dl, y1 = _tc_prep(hp, degl)

    y1v = y1.reshape(2 * NPAD, 64)
    z1a = _sc_aggregate(y1v, src_p, dst_p, 64, idx_mul=2, idx_off=0)
    z1b = _sc_aggregate(y1v, src_p, dst_p, 64, idx_mul=2, idx_off=1)
    zp1 = (z1a.reshape(2, LPAD, 128), z1b.reshape(2, LPAD, 128))
    y2 = _tc_stage(zp1, y1, dl, b1r, g1r, be1r, a2, W2)
    z2 = _sc_aggregate(y2.reshape(NPAD, 64), src_p, dst_p, 64)
    y3 = _tc_stage((z2.reshape(2, LPAD, 128),), y2, dl, b2r, g2r, be2r, a2, W3p)
    z3 = _sc_aggregate(y3.reshape(NPAD, 64), src_p, dst_p, 64)
    y4 = _tc_stage((z3.reshape(2, LPAD, 128),), y3, dl, b3p, g3p, be3p, a2, W4p)
    z4 = _sc_aggregate(y4.reshape(NPAD, 64), src_p, dst_p, 64)
    out = _tc_final(z4.reshape(2, LPAD, 128), y4, dl, b4p)
    return out.reshape(N, D_OUT)
